# baseline (speedup 1.0000x reference)
import functools

import jax
import jax.numpy as jnp
from jax import lax
from jax.experimental import pallas as pl
from jax.experimental.pallas import tpu as pltpu
from jax.experimental.pallas import tpu_sc as plsc

T = 200
B = 4096
D = 64
N = T * B
NC = 2
NS = 16
NW = NC * NS
NPW = N // NW
CHUNK = 1024
NCHUNK = NPW // CHUNK

_mesh = plsc.VectorSubcoreMesh(core_axis_name="c", subcore_axis_name="s")


@functools.partial(
    pl.kernel,
    out_type=jax.ShapeDtypeStruct((N, D), jnp.float32),
    mesh=_mesh,
    scratch_types=[
        pltpu.VMEM((CHUNK,), jnp.int32),
        pltpu.VMEM((CHUNK, D), jnp.float32),
        pltpu.SemaphoreType.DMA,
    ],
    compiler_params=pltpu.CompilerParams(use_tc_tiling_on_sc=False),
)
def _gather_kernel(idx_hbm, table_hbm, out_hbm, idx_v, rows_v, sem):
    wid = lax.axis_index("s") * NC + lax.axis_index("c")
    base = wid * NPW

    def body(i, carry):
        off = base + i * CHUNK
        pltpu.sync_copy(idx_hbm.at[pl.ds(off, CHUNK)], idx_v)
        pltpu.async_copy(table_hbm.at[idx_v], rows_v, sem).wait()
        pltpu.sync_copy(rows_v, out_hbm.at[pl.ds(off, CHUNK)])
        return carry

    lax.fori_loop(0, NCHUNK, body, 0)


def kernel(x, table):
    flat = x.reshape(-1)
    out = _gather_kernel(flat, table)
    return out.reshape(T, B, D)


# speedup vs baseline: 1.0124x; 1.0124x over previous
import functools

import jax
import jax.numpy as jnp
from jax import lax
from jax.experimental import pallas as pl
from jax.experimental.pallas import tpu as pltpu
from jax.experimental.pallas import tpu_sc as plsc

T = 200
B = 4096
D = 64
N = T * B
NC = 2
NS = 16
NW = NC * NS
NPW = N // NW
CHUNK = 800
NCHUNK = NPW // CHUNK
NSTEP = NCHUNK // 2

_mesh = plsc.VectorSubcoreMesh(core_axis_name="c", subcore_axis_name="s")


@functools.partial(
    pl.kernel,
    out_type=jax.ShapeDtypeStruct((N, D), jnp.float32),
    mesh=_mesh,
    scratch_types=[
        pltpu.VMEM((CHUNK,), jnp.int32),
        pltpu.VMEM((CHUNK,), jnp.int32),
        pltpu.VMEM((CHUNK, D), jnp.float32),
        pltpu.VMEM((CHUNK, D), jnp.float32),
        pltpu.SemaphoreType.DMA,
        pltpu.SemaphoreType.DMA,
        pltpu.SemaphoreType.DMA,
        pltpu.SemaphoreType.DMA,
        pltpu.SemaphoreType.DMA,
        pltpu.SemaphoreType.DMA,
    ],
    compiler_params=pltpu.CompilerParams(use_tc_tiling_on_sc=False),
)
def _gather_kernel(idx_hbm, table_hbm, out_hbm, idx0, idx1, rows0, rows1,
                   isem0, isem1, gsem0, gsem1, wsem0, wsem1):
    wid = lax.axis_index("s") * NC + lax.axis_index("c")
    base = wid * NPW

    def start_idx(buf, sem, chunk):
        off = base + lax.min(chunk, NCHUNK - 1) * CHUNK
        pltpu.async_copy(idx_hbm.at[pl.ds(off, CHUNK)], buf, sem)

    def wait_idx(buf, sem):
        pltpu.make_async_copy(idx_hbm.at[pl.ds(base, CHUNK)], buf, sem).wait()

    def start_gather(ibuf, rbuf, sem):
        return pltpu.async_copy(table_hbm.at[ibuf], rbuf, sem)

    def start_wb(rbuf, sem, chunk):
        off = base + chunk * CHUNK
        pltpu.async_copy(rbuf, out_hbm.at[pl.ds(off, CHUNK)], sem)

    def wait_wb(rbuf, sem):
        pltpu.make_async_copy(rbuf, out_hbm.at[pl.ds(base, CHUNK)], sem).wait()

    start_idx(idx0, isem0, 0)
    start_idx(idx1, isem1, 1)
    wait_idx(idx0, isem0)
    g0 = start_gather(idx0, rows0, gsem0)
    wait_idx(idx1, isem1)
    g1 = start_gather(idx1, rows1, gsem1)
    g0.wait()
    start_wb(rows0, wsem0, 0)
    start_idx(idx0, isem0, 2)
    g1.wait()
    start_wb(rows1, wsem1, 1)
    start_idx(idx1, isem1, 3)

    def body(s, carry):
        c0 = 2 * s
        wait_idx(idx0, isem0)
        wait_wb(rows0, wsem0)
        d0 = start_gather(idx0, rows0, gsem0)
        wait_idx(idx1, isem1)
        wait_wb(rows1, wsem1)
        d1 = start_gather(idx1, rows1, gsem1)
        d0.wait()
        start_wb(rows0, wsem0, c0)
        start_idx(idx0, isem0, c0 + 2)
        d1.wait()
        start_wb(rows1, wsem1, c0 + 1)
        start_idx(idx1, isem1, c0 + 3)
        return carry

    lax.fori_loop(1, NSTEP, body, 0)

    wait_wb(rows0, wsem0)
    wait_wb(rows1, wsem1)
    wait_idx(idx0, isem0)
    wait_idx(idx1, isem1)


def kernel(x, table):
    flat = x.reshape(-1)
    out = _gather_kernel(flat, table)
    return out.reshape(T, B, D)
